# Initial kernel scaffold; baseline (speedup 1.0000x reference)
import jax
import jax.numpy as jnp
from jax.experimental import pallas as pl


def kernel(x, tables):
    raise NotImplementedError("write your pallas kernel here")


# speedup vs baseline: 22.0024x; 22.0024x over previous
import functools

import jax
import jax.numpy as jnp
from jax import lax
from jax.experimental import pallas as pl
from jax.experimental.pallas import tpu as pltpu, tpu_sc as plsc

N_PTS = 262144
N_LVL = 16
N_FEAT = 2
HASH = 2 ** 19
MASK = HASH - 1
RES = [16, 20, 25, 32, 40, 50, 64, 80, 101, 128, 161, 203, 256, 322, 406, 512]
P1 = -1640531535
P2 = 805459861

NC, NS, LANES = 2, 16, 16
NW = NC * NS
PPW = N_PTS // NW
C = 512
NCH = PPW // C


def _minmax_body(x_ref, mm_ref):
    i = pl.program_id(0)
    xb = x_ref[...]
    mn = jnp.min(xb, axis=0, keepdims=True)
    mx = jnp.max(xb, axis=0, keepdims=True)

    @pl.when(i == 0)
    def _():
        mm_ref[0:1, :] = mn
        mm_ref[1:2, :] = mx

    @pl.when(i > 0)
    def _():
        mm_ref[0:1, :] = jnp.minimum(mm_ref[0:1, :], mn)
        mm_ref[1:2, :] = jnp.maximum(mm_ref[1:2, :], mx)

    @pl.when(i == pl.num_programs(0) - 1)
    def _():
        mm_ref[1:2, :] = jnp.float32(1.0) / (
            mm_ref[1:2, :] - mm_ref[0:1, :] + jnp.float32(1e-8))


def _minmax(x):
    blk = 8192
    return pl.pallas_call(
        _minmax_body,
        grid=(N_PTS // blk,),
        in_specs=[pl.BlockSpec((blk, 3), lambda i: (i, jnp.int32(0)))],
        out_specs=pl.BlockSpec((2, 3), lambda i: (jnp.int32(0), jnp.int32(0))),
        out_shape=jax.ShapeDtypeStruct((2, 3), jnp.float32),
    )(x)


def _sc_body(x_hbm, mn_hbm, inv_hbm, tab_hbm, out_hbm,
             xr_v, xs_v, fr_v, idx_v, lo_v, rows_v, out_v, cn_v, iv_v, sem):
    wid = lax.axis_index("s") * NC + lax.axis_index("c")
    base = wid * jnp.int32(PPW)
    pltpu.sync_copy(mn_hbm, cn_v)
    pltpu.sync_copy(inv_hbm, iv_v)
    iot = lax.iota(jnp.int32, LANES)
    lane0 = jnp.full((LANES,), 0, jnp.int32)
    lane1 = jnp.full((LANES,), 1, jnp.int32)
    dsel = [lane0, lane1, jnp.full((LANES,), 2, jnp.int32)]
    csel = [lane1, dsel[2], jnp.full((LANES,), 3, jnp.int32)]
    mn = [plsc.load_gather(cn_v, [csel[d]]) for d in range(3)]
    iv = [plsc.load_gather(iv_v, [csel[d]]) for d in range(3)]

    def chunk(s, carry):
        cbase = base + s * jnp.int32(C)
        pltpu.sync_copy(x_hbm.at[pl.ds(cbase, C)], xr_v)

        def p0(i, c2):
            p = i * jnp.int32(LANES)
            pv = p + iot
            for d in range(3):
                xd = plsc.load_gather(xr_v, [pv, dsel[d]])
                xs_v[pl.ds(jnp.int32(d * C) + p, LANES)] = (xd - mn[d]) * iv[d]
            return c2

        lax.fori_loop(jnp.int32(0), jnp.int32(C // LANES), p0, jnp.int32(0))

        for lv in range(N_LVL):
            res = jnp.float32(RES[lv])
            off = lv * HASH

            def pa(i, c2, res=res, off=off):
                p = i * jnp.int32(LANES)
                xg = [xs_v[pl.ds(jnp.int32(d * C) + p, LANES)] * res for d in range(3)]
                xf = [g.astype(jnp.int32) for g in xg]
                for d in range(3):
                    fr_v[pl.ds(jnp.int32(d * C) + p, LANES)] = (
                        xg[d] - xf[d].astype(jnp.float32))
                a0 = xf[0]
                b0 = a0 + jnp.int32(1)
                a1 = xf[1] * jnp.int32(P1)
                b1 = a1 + jnp.int32(P1)
                a2 = xf[2] * jnp.int32(P2)
                b2 = a2 + jnp.int32(P2)
                t = [a0 ^ a1, b0 ^ a1, a0 ^ b1, b0 ^ b1]
                for c in range(8):
                    h = (t[c & 3] ^ (a2 if c < 4 else b2)) & jnp.int32(MASK)
                    full = h + jnp.int32(off)
                    idx_v[pl.ds(jnp.int32(c * C) + p, LANES)] = (
                        lax.shift_right_logical(full, jnp.int32(2)))
                    lo_v[pl.ds(jnp.int32(c * C) + p, LANES)] = (
                        lax.shift_left(full & jnp.int32(3), jnp.int32(1)))
                return c2

            lax.fori_loop(jnp.int32(0), jnp.int32(C // LANES), pa, jnp.int32(0))
            pltpu.async_copy(tab_hbm.at[idx_v], rows_v, sem).wait()

            def pb(i, c2, lv=lv):
                p = i * jnp.int32(LANES)
                pv = p + iot
                fr = [fr_v[pl.ds(jnp.int32(d * C) + p, LANES)] for d in range(3)]
                u = [jnp.float32(1.0) - f for f in fr]
                wxy = [u[0] * u[1], fr[0] * u[1], u[0] * fr[1], fr[0] * fr[1]]
                acc0 = jnp.zeros((LANES,), jnp.float32)
                acc1 = jnp.zeros((LANES,), jnp.float32)
                for c in range(8):
                    wc = wxy[c & 3] * (u[2] if c < 4 else fr[2])
                    lo = lo_v[pl.ds(jnp.int32(c * C) + p, LANES)]
                    f0 = plsc.load_gather(rows_v, [jnp.int32(c * C) + pv, lo])
                    f1 = plsc.load_gather(
                        rows_v, [jnp.int32(c * C) + pv, lo + jnp.int32(1)])
                    acc0 = acc0 + wc * f0
                    acc1 = acc1 + wc * f1
                plsc.store_scatter(
                    out_v, [pv, jnp.full((LANES,), 2 * lv, jnp.int32)], acc0)
                plsc.store_scatter(
                    out_v, [pv, jnp.full((LANES,), 2 * lv + 1, jnp.int32)], acc1)
                return c2

            lax.fori_loop(jnp.int32(0), jnp.int32(C // LANES), pb, jnp.int32(0))

        pltpu.sync_copy(out_v, out_hbm.at[pl.ds(cbase, C)])
        return carry

    lax.fori_loop(jnp.int32(0), jnp.int32(NCH), chunk, jnp.int32(0))


def _sc_call(x, mn16, inv16, tab_flat):
    mesh = plsc.VectorSubcoreMesh(
        core_axis_name="c", subcore_axis_name="s",
        num_cores=NC, num_subcores=NS)
    return pl.kernel(
        _sc_body,
        out_type=jax.ShapeDtypeStruct((N_PTS, 2 * N_LVL), jnp.float32),
        mesh=mesh,
        compiler_params=pltpu.CompilerParams(needs_layout_passes=False, use_tc_tiling_on_sc=False),
        scratch_types=[
            pltpu.VMEM((C, 3), jnp.float32),
            pltpu.VMEM((3 * C,), jnp.float32),
            pltpu.VMEM((3 * C,), jnp.float32),
            pltpu.VMEM((8 * C,), jnp.int32),
            pltpu.VMEM((8 * C,), jnp.int32),
            pltpu.VMEM((8 * C, 8), jnp.float32),
            pltpu.VMEM((C, 2 * N_LVL), jnp.float32),
            pltpu.VMEM((LANES,), jnp.float32),
            pltpu.VMEM((LANES,), jnp.float32),
            pltpu.SemaphoreType.DMA,
        ],
    )(x, mn16, inv16, tab_flat)


@jax.jit
def kernel(x, tables):
    x = x.astype(jnp.float32)
    mm = _minmax(x)
    pad = jnp.concatenate(
        [jnp.ones((2, 1), jnp.float32), mm, jnp.ones((2, 12), jnp.float32)],
        axis=1)
    mn16 = pad[0]
    inv16 = pad[1]
    tab_flat = tables.astype(jnp.float32).reshape(N_LVL * HASH * N_FEAT // 8, 8)
    return _sc_call(x, mn16, inv16, tab_flat)


# speedup vs baseline: 23.1944x; 1.0542x over previous
import jax
import jax.numpy as jnp
from jax import lax
from jax.experimental import pallas as pl
from jax.experimental.pallas import tpu as pltpu, tpu_sc as plsc

N_PTS = 262144
N_LVL = 16
N_FEAT = 2
HASH = 2 ** 19
MASK = HASH - 1
RES = [16, 20, 25, 32, 40, 50, 64, 80, 101, 128, 161, 203, 256, 322, 406, 512]
P1 = -1640531535
P2 = 805459861

NC, NS, LANES = 2, 16, 16
NW = NC * NS
PPW = N_PTS // NW
C = 512
NCH = PPW // C


def _minmax_body(x_ref, mm_ref):
    i = pl.program_id(0)
    xb = x_ref[...]
    mn = jnp.min(xb, axis=0, keepdims=True)
    mx = jnp.max(xb, axis=0, keepdims=True)

    @pl.when(i == 0)
    def _():
        mm_ref[0:1, :] = mn
        mm_ref[1:2, :] = mx

    @pl.when(i > 0)
    def _():
        mm_ref[0:1, :] = jnp.minimum(mm_ref[0:1, :], mn)
        mm_ref[1:2, :] = jnp.maximum(mm_ref[1:2, :], mx)

    @pl.when(i == pl.num_programs(0) - 1)
    def _():
        mm_ref[1:2, :] = jnp.float32(1.0) / (
            mm_ref[1:2, :] - mm_ref[0:1, :] + jnp.float32(1e-8))


def _minmax(x):
    blk = 8192
    return pl.pallas_call(
        _minmax_body,
        grid=(N_PTS // blk,),
        in_specs=[pl.BlockSpec((blk, 3), lambda i: (i, jnp.int32(0)))],
        out_specs=pl.BlockSpec((2, 3), lambda i: (jnp.int32(0), jnp.int32(0))),
        out_shape=jax.ShapeDtypeStruct((2, 3), jnp.float32),
    )(x)


def _sc_body(x_hbm, mn_hbm, inv_hbm, tab_hbm, out_hbm,
             xr_v, xs_v, out_v, cn_v, iv_v,
             fr0, fr1, idx0, idx1, lo0, lo1, rows0, rows1, sem0, sem1):
    frs = [fr0, fr1]
    idxs = [idx0, idx1]
    los = [lo0, lo1]
    rowss = [rows0, rows1]
    sems = [sem0, sem1]

    wid = lax.axis_index("s") * NC + lax.axis_index("c")
    base = wid * jnp.int32(PPW)
    pltpu.sync_copy(mn_hbm, cn_v)
    pltpu.sync_copy(inv_hbm, iv_v)
    iot = lax.iota(jnp.int32, LANES)
    lane0 = jnp.full((LANES,), 0, jnp.int32)
    lane1 = jnp.full((LANES,), 1, jnp.int32)
    dsel = [lane0, lane1, jnp.full((LANES,), 2, jnp.int32)]
    csel = [lane1, dsel[2], jnp.full((LANES,), 3, jnp.int32)]
    mn = [plsc.load_gather(cn_v, [csel[d]]) for d in range(3)]
    iv = [plsc.load_gather(iv_v, [csel[d]]) for d in range(3)]

    def pass_a(lv, b):
        res = jnp.float32(RES[lv])
        off = lv * HASH
        fr_v, idx_v, lo_v = frs[b], idxs[b], los[b]

        def pa(i, c2):
            p = i * jnp.int32(LANES)
            xg = [xs_v[pl.ds(jnp.int32(d * C) + p, LANES)] * res
                  for d in range(3)]
            xf = [g.astype(jnp.int32) for g in xg]
            for d in range(3):
                fr_v[pl.ds(jnp.int32(d * C) + p, LANES)] = (
                    xg[d] - xf[d].astype(jnp.float32))
            a0 = xf[0]
            b0 = a0 + jnp.int32(1)
            a1 = xf[1] * jnp.int32(P1)
            b1 = a1 + jnp.int32(P1)
            a2 = xf[2] * jnp.int32(P2)
            b2 = a2 + jnp.int32(P2)
            t = [a0 ^ a1, b0 ^ a1, a0 ^ b1, b0 ^ b1]
            for c in range(8):
                h = (t[c & 3] ^ (a2 if c < 4 else b2)) & jnp.int32(MASK)
                full = h + jnp.int32(off)
                idx_v[pl.ds(jnp.int32(c * C) + p, LANES)] = (
                    lax.shift_right_logical(full, jnp.int32(2)))
                lo_v[pl.ds(jnp.int32(c * C) + p, LANES)] = (
                    lax.shift_left(full & jnp.int32(3), jnp.int32(1)))
            return c2

        lax.fori_loop(jnp.int32(0), jnp.int32(C // LANES), pa, jnp.int32(0))

    def fire(b):
        return pltpu.async_copy(tab_hbm.at[idxs[b]], rowss[b], sems[b])

    def pass_b(lv, b):
        fr_v, lo_v, rows_v = frs[b], los[b], rowss[b]

        def pb(i, c2):
            p = i * jnp.int32(LANES)
            pv = p + iot
            fr = [fr_v[pl.ds(jnp.int32(d * C) + p, LANES)] for d in range(3)]
            u = [jnp.float32(1.0) - f for f in fr]
            wxy = [u[0] * u[1], fr[0] * u[1], u[0] * fr[1], fr[0] * fr[1]]
            acc0 = jnp.zeros((LANES,), jnp.float32)
            acc1 = jnp.zeros((LANES,), jnp.float32)
            for c in range(8):
                wc = wxy[c & 3] * (u[2] if c < 4 else fr[2])
                lo = lo_v[pl.ds(jnp.int32(c * C) + p, LANES)]
                f0 = plsc.load_gather(rows_v, [jnp.int32(c * C) + pv, lo])
                f1 = plsc.load_gather(
                    rows_v, [jnp.int32(c * C) + pv, lo + jnp.int32(1)])
                acc0 = acc0 + wc * f0
                acc1 = acc1 + wc * f1
            plsc.store_scatter(
                out_v, [pv, jnp.full((LANES,), 2 * lv, jnp.int32)], acc0)
            plsc.store_scatter(
                out_v, [pv, jnp.full((LANES,), 2 * lv + 1, jnp.int32)], acc1)
            return c2

        lax.fori_loop(jnp.int32(0), jnp.int32(C // LANES), pb, jnp.int32(0))

    def chunk(s, carry):
        cbase = base + s * jnp.int32(C)
        pltpu.sync_copy(x_hbm.at[pl.ds(cbase, C)], xr_v)

        def p0(i, c2):
            p = i * jnp.int32(LANES)
            pv = p + iot
            for d in range(3):
                xd = plsc.load_gather(xr_v, [pv, dsel[d]])
                xs_v[pl.ds(jnp.int32(d * C) + p, LANES)] = (xd - mn[d]) * iv[d]
            return c2

        lax.fori_loop(jnp.int32(0), jnp.int32(C // LANES), p0, jnp.int32(0))

        handles = [None, None]
        pass_a(0, 0)
        handles[0] = fire(0)
        for lv in range(N_LVL):
            b = lv % 2
            if lv + 1 < N_LVL:
                pass_a(lv + 1, 1 - b)
                handles[1 - b] = fire(1 - b)
            handles[b].wait()
            pass_b(lv, b)

        pltpu.sync_copy(out_v, out_hbm.at[pl.ds(cbase, C)])
        return carry

    lax.fori_loop(jnp.int32(0), jnp.int32(NCH), chunk, jnp.int32(0))


def _sc_call(x, mn16, inv16, tab_flat):
    mesh = plsc.VectorSubcoreMesh(
        core_axis_name="c", subcore_axis_name="s",
        num_cores=NC, num_subcores=NS)
    return pl.kernel(
        _sc_body,
        out_type=jax.ShapeDtypeStruct((N_PTS, 2 * N_LVL), jnp.float32),
        mesh=mesh,
        compiler_params=pltpu.CompilerParams(
            needs_layout_passes=False, use_tc_tiling_on_sc=False),
        scratch_types=[
            pltpu.VMEM((C, 3), jnp.float32),
            pltpu.VMEM((3 * C,), jnp.float32),
            pltpu.VMEM((C, 2 * N_LVL), jnp.float32),
            pltpu.VMEM((LANES,), jnp.float32),
            pltpu.VMEM((LANES,), jnp.float32),
            pltpu.VMEM((3 * C,), jnp.float32),
            pltpu.VMEM((3 * C,), jnp.float32),
            pltpu.VMEM((8 * C,), jnp.int32),
            pltpu.VMEM((8 * C,), jnp.int32),
            pltpu.VMEM((8 * C,), jnp.int32),
            pltpu.VMEM((8 * C,), jnp.int32),
            pltpu.VMEM((8 * C, 8), jnp.float32),
            pltpu.VMEM((8 * C, 8), jnp.float32),
            pltpu.SemaphoreType.DMA,
            pltpu.SemaphoreType.DMA,
        ],
    )(x, mn16, inv16, tab_flat)


@jax.jit
def kernel(x, tables):
    x = x.astype(jnp.float32)
    mm = _minmax(x)
    pad = jnp.concatenate(
        [jnp.ones((2, 1), jnp.float32), mm, jnp.ones((2, 12), jnp.float32)],
        axis=1)
    mn16 = pad[0]
    inv16 = pad[1]
    tab_flat = tables.astype(jnp.float32).reshape(N_LVL * HASH * N_FEAT // 8, 8)
    return _sc_call(x, mn16, inv16, tab_flat)


# speedup vs baseline: 23.2325x; 1.0016x over previous
import jax
import jax.numpy as jnp
from jax import lax
from jax.experimental import pallas as pl
from jax.experimental.pallas import tpu as pltpu, tpu_sc as plsc

N_PTS = 262144
N_LVL = 16
N_FEAT = 2
HASH = 2 ** 19
MASK = HASH - 1
RES = [16, 20, 25, 32, 40, 50, 64, 80, 101, 128, 161, 203, 256, 322, 406, 512]
P1 = -1640531535
P2 = 805459861

NC, NS, LANES = 2, 16, 16
NW = NC * NS
PPW = N_PTS // NW
C = 256
NCH = PPW // C
NBUF = 4
DEPTH = 3


def _minmax_body(x_ref, mm_ref):
    i = pl.program_id(0)
    xb = x_ref[...]
    mn = jnp.min(xb, axis=0, keepdims=True)
    mx = jnp.max(xb, axis=0, keepdims=True)

    @pl.when(i == 0)
    def _():
        mm_ref[0:1, :] = mn
        mm_ref[1:2, :] = mx

    @pl.when(i > 0)
    def _():
        mm_ref[0:1, :] = jnp.minimum(mm_ref[0:1, :], mn)
        mm_ref[1:2, :] = jnp.maximum(mm_ref[1:2, :], mx)

    @pl.when(i == pl.num_programs(0) - 1)
    def _():
        mm_ref[1:2, :] = jnp.float32(1.0) / (
            mm_ref[1:2, :] - mm_ref[0:1, :] + jnp.float32(1e-8))


def _minmax(x):
    blk = 8192
    return pl.pallas_call(
        _minmax_body,
        grid=(N_PTS // blk,),
        in_specs=[pl.BlockSpec((blk, 3), lambda i: (i, jnp.int32(0)))],
        out_specs=pl.BlockSpec((2, 3), lambda i: (jnp.int32(0), jnp.int32(0))),
        out_shape=jax.ShapeDtypeStruct((2, 3), jnp.float32),
    )(x)


def _sc_body(x_hbm, mn_hbm, inv_hbm, tab_hbm, out_hbm,
             xr_v, xs_v, out_v, cn_v, iv_v, *bufs):
    frs = list(bufs[0:NBUF])
    idxs = list(bufs[NBUF:2 * NBUF])
    los = list(bufs[2 * NBUF:3 * NBUF])
    rowss = list(bufs[3 * NBUF:4 * NBUF])
    sems = list(bufs[4 * NBUF:5 * NBUF])

    wid = lax.axis_index("s") * NC + lax.axis_index("c")
    base = wid * jnp.int32(PPW)
    pltpu.sync_copy(mn_hbm, cn_v)
    pltpu.sync_copy(inv_hbm, iv_v)
    iot = lax.iota(jnp.int32, LANES)
    lane0 = jnp.full((LANES,), 0, jnp.int32)
    lane1 = jnp.full((LANES,), 1, jnp.int32)
    dsel = [lane0, lane1, jnp.full((LANES,), 2, jnp.int32)]
    csel = [lane1, dsel[2], jnp.full((LANES,), 3, jnp.int32)]
    mn = [plsc.load_gather(cn_v, [csel[d]]) for d in range(3)]
    iv = [plsc.load_gather(iv_v, [csel[d]]) for d in range(3)]

    def load_chunk(s):
        pltpu.sync_copy(x_hbm.at[pl.ds(base + s * jnp.int32(C), C)], xr_v)

        def p0(i, c2):
            p = i * jnp.int32(LANES)
            pv = p + iot
            for d in range(3):
                xd = plsc.load_gather(xr_v, [pv, dsel[d]])
                xs_v[pl.ds(jnp.int32(d * C) + p, LANES)] = (xd - mn[d]) * iv[d]
            return c2

        lax.fori_loop(jnp.int32(0), jnp.int32(C // LANES), p0, jnp.int32(0))

    def pass_a(lv, b):
        res = jnp.float32(RES[lv])
        off = lv * HASH
        fr_v, idx_v, lo_v = frs[b], idxs[b], los[b]

        def pa(i, c2):
            p = i * jnp.int32(LANES)
            xg = [xs_v[pl.ds(jnp.int32(d * C) + p, LANES)] * res
                  for d in range(3)]
            xf = [g.astype(jnp.int32) for g in xg]
            for d in range(3):
                fr_v[pl.ds(jnp.int32(d * C) + p, LANES)] = (
                    xg[d] - xf[d].astype(jnp.float32))
            a0 = xf[0]
            b0 = a0 + jnp.int32(1)
            a1 = xf[1] * jnp.int32(P1)
            b1 = a1 + jnp.int32(P1)
            a2 = xf[2] * jnp.int32(P2)
            b2 = a2 + jnp.int32(P2)
            t = [a0 ^ a1, b0 ^ a1, a0 ^ b1, b0 ^ b1]
            for c in range(8):
                h = (t[c & 3] ^ (a2 if c < 4 else b2)) & jnp.int32(MASK)
                full = h + jnp.int32(off)
                idx_v[pl.ds(jnp.int32(c * C) + p, LANES)] = (
                    lax.shift_right_logical(full, jnp.int32(2)))
                lo_v[pl.ds(jnp.int32(c * C) + p, LANES)] = (
                    lax.shift_left(full & jnp.int32(3), jnp.int32(1)))
            return c2

        lax.fori_loop(jnp.int32(0), jnp.int32(C // LANES), pa, jnp.int32(0))

    def fire(b):
        pltpu.async_copy(tab_hbm.at[idxs[b]], rowss[b], sems[b])

    def wait(b):
        pltpu.make_async_copy(tab_hbm.at[idxs[b]], rowss[b], sems[b]).wait()

    def pass_b(lv, b):
        fr_v, lo_v, rows_v = frs[b], los[b], rowss[b]

        def pb(i, c2):
            p = i * jnp.int32(LANES)
            pv = p + iot
            fr = [fr_v[pl.ds(jnp.int32(d * C) + p, LANES)] for d in range(3)]
            u = [jnp.float32(1.0) - f for f in fr]
            wxy = [u[0] * u[1], fr[0] * u[1], u[0] * fr[1], fr[0] * fr[1]]
            acc0 = jnp.zeros((LANES,), jnp.float32)
            acc1 = jnp.zeros((LANES,), jnp.float32)
            for c in range(8):
                wc = wxy[c & 3] * (u[2] if c < 4 else fr[2])
                lo = lo_v[pl.ds(jnp.int32(c * C) + p, LANES)]
                f0 = plsc.load_gather(rows_v, [jnp.int32(c * C) + pv, lo])
                f1 = plsc.load_gather(
                    rows_v, [jnp.int32(c * C) + pv, lo + jnp.int32(1)])
                acc0 = acc0 + wc * f0
                acc1 = acc1 + wc * f1
            plsc.store_scatter(
                out_v, [pv, jnp.full((LANES,), 2 * lv, jnp.int32)], acc0)
            plsc.store_scatter(
                out_v, [pv, jnp.full((LANES,), 2 * lv + 1, jnp.int32)], acc1)
            return c2

        lax.fori_loop(jnp.int32(0), jnp.int32(C // LANES), pb, jnp.int32(0))

    load_chunk(jnp.int32(0))
    for lv in range(DEPTH):
        pass_a(lv, lv % NBUF)
        fire(lv % NBUF)

    def chunk(s, carry):
        for lv in range(N_LVL):
            b = lv % NBUF
            nb = (lv + DEPTH) % NBUF
            if lv + DEPTH < N_LVL:
                pass_a(lv + DEPTH, nb)
                fire(nb)
            else:
                nxt = lv + DEPTH - N_LVL

                @pl.when(s + jnp.int32(1) < jnp.int32(NCH))
                def _(nxt=nxt, nb=nb):
                    if nxt == 0:
                        load_chunk(s + jnp.int32(1))
                    pass_a(nxt, nb)
                    fire(nb)

            wait(b)
            pass_b(lv, b)

        pltpu.sync_copy(
            out_v, out_hbm.at[pl.ds(base + s * jnp.int32(C), C)])
        return carry

    lax.fori_loop(jnp.int32(0), jnp.int32(NCH), chunk, jnp.int32(0))


def _sc_call(x, mn16, inv16, tab_flat):
    mesh = plsc.VectorSubcoreMesh(
        core_axis_name="c", subcore_axis_name="s",
        num_cores=NC, num_subcores=NS)
    scratch = [
        pltpu.VMEM((C, 3), jnp.float32),
        pltpu.VMEM((3 * C,), jnp.float32),
        pltpu.VMEM((C, 2 * N_LVL), jnp.float32),
        pltpu.VMEM((LANES,), jnp.float32),
        pltpu.VMEM((LANES,), jnp.float32),
    ]
    scratch += [pltpu.VMEM((3 * C,), jnp.float32) for _ in range(NBUF)]
    scratch += [pltpu.VMEM((8 * C,), jnp.int32) for _ in range(NBUF)]
    scratch += [pltpu.VMEM((8 * C,), jnp.int32) for _ in range(NBUF)]
    scratch += [pltpu.VMEM((8 * C, 8), jnp.float32) for _ in range(NBUF)]
    scratch += [pltpu.SemaphoreType.DMA for _ in range(NBUF)]
    return pl.kernel(
        _sc_body,
        out_type=jax.ShapeDtypeStruct((N_PTS, 2 * N_LVL), jnp.float32),
        mesh=mesh,
        compiler_params=pltpu.CompilerParams(
            needs_layout_passes=False, use_tc_tiling_on_sc=False),
        scratch_types=scratch,
    )(x, mn16, inv16, tab_flat)


@jax.jit
def kernel(x, tables):
    x = x.astype(jnp.float32)
    mm = _minmax(x)
    pad = jnp.concatenate(
        [jnp.ones((2, 1), jnp.float32), mm, jnp.ones((2, 12), jnp.float32)],
        axis=1)
    mn16 = pad[0]
    inv16 = pad[1]
    tab_flat = tables.astype(jnp.float32).reshape(N_LVL * HASH * N_FEAT // 8, 8)
    return _sc_call(x, mn16, inv16, tab_flat)
